# Initial kernel scaffold; baseline (speedup 1.0000x reference)
import functools

import jax
import jax.numpy as jnp
from jax import lax
from jax.experimental import pallas as pl
from jax.experimental.pallas import tpu as pltpu
from jax.experimental.pallas import tpu_sc as plsc

NC = 2
NS = 16
L = 16

NP = 10240
DH = 128
CHUNK = 128
C_CHUNKS = 79
EP = NS * C_CHUNKS * CHUNK
RPW = NP // (NC * NS)
RPT = NP // NS

_mesh = plsc.VectorSubcoreMesh(
    core_axis_name="c", subcore_axis_name="s", num_cores=NC, num_subcores=NS)


def _bcast_lane(v, lane):
    idx = jnp.full((L,), lane, dtype=jnp.int32)
    return v.at[idx].get(mode="promise_in_bounds")


def _rsqrt(m):
    i = plsc.bitcast(m, jnp.int32)
    i = jnp.int32(0x5F3759DF) - lax.shift_right_logical(i, 1)
    y = plsc.bitcast(i, jnp.float32)
    for _ in range(3):
        y = y * (1.5 - 0.5 * m * y * y)
    return y


@functools.partial(
    pl.kernel,
    out_type=jax.ShapeDtypeStruct((2 * NP, DH), jnp.float32),
    mesh=_mesh,
    scratch_types=[
        pltpu.MemorySpace.VMEM_SHARED((NP, DH), jnp.float32),
        pltpu.VMEM((C_CHUNKS, CHUNK), jnp.int32),
        pltpu.VMEM((C_CHUNKS, CHUNK), jnp.int32),
        pltpu.VMEM((C_CHUNKS, CHUNK), jnp.float32),
        pltpu.VMEM((CHUNK, DH), jnp.float32),
        pltpu.SemaphoreType.DMA,
    ],
)
def _spmm(x_hbm, src_hbm, dst_hbm, w_hbm, sup_hbm,
          acc, src_v, dst_v, w_v, rows_v, sem):
    c = lax.axis_index("c")
    s = lax.axis_index("s")

    zero = jnp.zeros((L,), jnp.float32)

    def zrow(j, carry):
        for g in range(DH // L):
            rows_v[j, pl.ds(g * L, L)] = zero
        return carry

    lax.fori_loop(0, CHUNK, zrow, 0)
    for t in range(RPT // CHUNK):
        pltpu.sync_copy(rows_v, acc.at[pl.ds(s * RPT + t * CHUNK, CHUNK)])
    plsc.subcore_barrier()

    pltpu.sync_copy(src_hbm.at[s], src_v)
    pltpu.sync_copy(dst_hbm.at[s], dst_v)
    pltpu.sync_copy(w_hbm.at[s], w_v)
    off = jnp.full((L,), c * NP, dtype=jnp.int32)

    def adj(j, carry):
        for g in range(CHUNK // L):
            src_v[j, pl.ds(g * L, L)] = src_v[j, pl.ds(g * L, L)] + off
        return carry

    lax.fori_loop(0, C_CHUNKS, adj, 0)

    def chunk_body(j, carry):
        pltpu.async_copy(x_hbm.at[src_v.at[j]], rows_v, sem).wait()

        def grp(i, carry2):
            wv = w_v[j, pl.ds(i * L, L)]
            for e in range(L):
                wb = _bcast_lane(wv, e)
                r = i * L + e
                for g in range(DH // L):
                    rows_v[r, pl.ds(g * L, L)] = rows_v[r, pl.ds(g * L, L)] * wb
            return carry2

        lax.fori_loop(0, CHUNK // L, grp, 0)
        pltpu.sync_copy(rows_v, acc.at[dst_v.at[j]], add=True)
        return carry

    lax.fori_loop(0, C_CHUNKS, chunk_body, 0)
    plsc.subcore_barrier()

    pltpu.sync_copy(acc.at[pl.ds(s * RPT, RPT)],
                    sup_hbm.at[pl.ds(c * NP + s * RPT, RPT)])


@functools.partial(
    pl.kernel,
    out_type=jax.ShapeDtypeStruct((2 * NP, DH), jnp.float32),
    mesh=_mesh,
    scratch_types=[
        pltpu.VMEM((RPW, DH), jnp.float32),
        pltpu.VMEM((RPW, DH), jnp.float32),
    ],
)
def _norm(sup_hbm, out_hbm, h0_v, h1_v):
    c = lax.axis_index("c")
    s = lax.axis_index("s")
    w = s * NC + c
    base = w * RPW
    pltpu.sync_copy(sup_hbm.at[pl.ds(base, RPW)], h0_v)
    pltpu.sync_copy(sup_hbm.at[pl.ds(NP + base, RPW)], h1_v)

    def row(r, carry):
        ss = jnp.zeros((L,), jnp.float32)
        v0 = h0_v[r, pl.ds(0, L)]
        for g in range(DH // L):
            v = h0_v[r, pl.ds(g * L, L)] if g else v0
            ss = ss + v * v
        for g in range(DH // L):
            v = h1_v[r, pl.ds(g * L, L)]
            ss = ss + v * v
        tot = _bcast_lane(plsc.cumsum(ss), L - 1)
        s0 = _bcast_lane(v0, 0)
        m = s0 * s0 * 2.0 - tot
        y = _rsqrt(jnp.maximum(jnp.abs(m), 1e-8))
        for g in range(DH // L):
            h0_v[r, pl.ds(g * L, L)] = h0_v[r, pl.ds(g * L, L)] * y
            h1_v[r, pl.ds(g * L, L)] = h1_v[r, pl.ds(g * L, L)] * y
        return carry

    lax.fori_loop(0, RPW, row, 0)
    pltpu.sync_copy(h0_v, out_hbm.at[pl.ds(base, RPW)])
    pltpu.sync_copy(h1_v, out_hbm.at[pl.ds(NP + base, RPW)])


def kernel(x, edge_index, edge_weight):
    n, d = x.shape
    src = edge_index[0]
    dst = edge_index[1]
    e = src.shape[0]

    x_cat = jnp.zeros((2 * NP, DH), x.dtype)
    x_cat = x_cat.at[:n].set(x[:, :DH]).at[NP:NP + n].set(x[:, DH:])

    pad = EP - e
    fill = jnp.arange(pad, dtype=jnp.int32) % n
    src_p = jnp.concatenate([src, fill]).reshape(NS, C_CHUNKS, CHUNK)
    dst_p = jnp.concatenate([dst, fill]).reshape(NS, C_CHUNKS, CHUNK)
    w_p = jnp.concatenate(
        [edge_weight, jnp.zeros((pad,), edge_weight.dtype)]
    ).reshape(NS, C_CHUNKS, CHUNK)

    h = x_cat
    for _ in range(2):
        sup = _spmm(h, src_p, dst_p, w_p)
        h = _norm(sup)
    return jnp.concatenate([h[:n], h[NP:NP + n]], axis=1)


# speedup vs baseline: 4.4067x; 4.4067x over previous
import functools

import jax
import jax.numpy as jnp
from jax import lax
from jax.experimental import pallas as pl
from jax.experimental.pallas import tpu as pltpu
from jax.experimental.pallas import tpu_sc as plsc

NC = 2
NS = 16
L = 16

NP = 10240
DH = 128
CHUNK = 128
C_CHUNKS = 79
EP = NS * C_CHUNKS * CHUNK
RPW = NP // (NC * NS)
RPT = NP // NS

_mesh = plsc.VectorSubcoreMesh(
    core_axis_name="c", subcore_axis_name="s", num_cores=NC, num_subcores=NS)


def _bcast_lane(v, lane):
    idx = jnp.full((L,), lane, dtype=jnp.int32)
    return v.at[idx].get(mode="promise_in_bounds")


def _lane_sum(v):
    iota = lax.iota(jnp.int32, L)
    for k in (8, 4, 2, 1):
        idx = lax.bitwise_xor(iota, k)
        v = v + v.at[idx].get(mode="promise_in_bounds")
    return v


def _rsqrt(m):
    i = plsc.bitcast(m, jnp.int32)
    i = jnp.int32(0x5F3759DF) - lax.shift_right_logical(i, 1)
    y = plsc.bitcast(i, jnp.float32)
    for _ in range(3):
        y = y * (1.5 - 0.5 * m * y * y)
    return y


@functools.partial(
    pl.kernel,
    out_type=jax.ShapeDtypeStruct((2 * NP, DH), jnp.float32),
    mesh=_mesh,
    scratch_types=[
        pltpu.MemorySpace.VMEM_SHARED((NP, DH), jnp.float32),
        pltpu.VMEM((C_CHUNKS, CHUNK), jnp.int32),
        pltpu.VMEM((C_CHUNKS, CHUNK), jnp.int32),
        pltpu.VMEM((C_CHUNKS, CHUNK), jnp.float32),
        pltpu.VMEM((CHUNK, DH), jnp.float32),
        pltpu.SemaphoreType.DMA,
    ],
    compiler_params=pltpu.CompilerParams(needs_layout_passes=False),
)
def _spmm(x_hbm, src_hbm, dst_hbm, w_hbm, sup_hbm,
          acc, src_v, dst_v, w_v, rows_v, sem):
    c = lax.axis_index("c")
    s = lax.axis_index("s")

    zero = jnp.zeros((L,), jnp.float32)

    def zrow(j, carry):
        for g in range(DH // L):
            rows_v[j, pl.ds(g * L, L)] = zero
        return carry

    lax.fori_loop(0, CHUNK, zrow, 0)
    for t in range(RPT // CHUNK):
        pltpu.sync_copy(rows_v, acc.at[pl.ds(s * RPT + t * CHUNK, CHUNK)])
    plsc.subcore_barrier()

    pltpu.sync_copy(src_hbm.at[s], src_v)
    pltpu.sync_copy(dst_hbm.at[s], dst_v)
    pltpu.sync_copy(w_hbm.at[s], w_v)
    off = jnp.full((L,), c * NP, dtype=jnp.int32)

    def adj(j, carry):
        for g in range(CHUNK // L):
            src_v[j, pl.ds(g * L, L)] = src_v[j, pl.ds(g * L, L)] + off
        return carry

    lax.fori_loop(0, C_CHUNKS, adj, 0)

    def chunk_body(j, carry):
        pltpu.async_copy(x_hbm.at[src_v.at[j]], rows_v, sem).wait()

        def grp(i, carry2):
            wv = w_v[j, pl.ds(i * L, L)]
            for e in range(L):
                wb = _bcast_lane(wv, e)
                r = i * L + e
                for g in range(DH // L):
                    rows_v[r, pl.ds(g * L, L)] = rows_v[r, pl.ds(g * L, L)] * wb
            return carry2

        lax.fori_loop(0, CHUNK // L, grp, 0)
        pltpu.sync_copy(rows_v, acc.at[dst_v.at[j]], add=True)
        return carry

    lax.fori_loop(0, C_CHUNKS, chunk_body, 0)
    plsc.subcore_barrier()

    pltpu.sync_copy(acc.at[pl.ds(s * RPT, RPT)],
                    sup_hbm.at[pl.ds(c * NP + s * RPT, RPT)])


@functools.partial(
    pl.kernel,
    out_type=jax.ShapeDtypeStruct((2 * NP, DH), jnp.float32),
    mesh=_mesh,
    scratch_types=[
        pltpu.VMEM((RPW, DH), jnp.float32),
        pltpu.VMEM((RPW, DH), jnp.float32),
    ],
    compiler_params=pltpu.CompilerParams(needs_layout_passes=False),
)
def _norm(sup_hbm, out_hbm, h0_v, h1_v):
    c = lax.axis_index("c")
    s = lax.axis_index("s")
    w = s * NC + c
    base = w * RPW
    pltpu.sync_copy(sup_hbm.at[pl.ds(base, RPW)], h0_v)
    pltpu.sync_copy(sup_hbm.at[pl.ds(NP + base, RPW)], h1_v)

    def row(r, carry):
        ss = jnp.zeros((L,), jnp.float32)
        v0 = h0_v[r, pl.ds(0, L)]
        for g in range(DH // L):
            v = h0_v[r, pl.ds(g * L, L)] if g else v0
            ss = ss + v * v
        for g in range(DH // L):
            v = h1_v[r, pl.ds(g * L, L)]
            ss = ss + v * v
        tot = _lane_sum(ss)
        s0 = _bcast_lane(v0, 0)
        m = s0 * s0 * 2.0 - tot
        y = _rsqrt(jnp.maximum(jnp.abs(m), 1e-8))
        for g in range(DH // L):
            h0_v[r, pl.ds(g * L, L)] = h0_v[r, pl.ds(g * L, L)] * y
            h1_v[r, pl.ds(g * L, L)] = h1_v[r, pl.ds(g * L, L)] * y
        return carry

    lax.fori_loop(0, RPW, row, 0)
    pltpu.sync_copy(h0_v, out_hbm.at[pl.ds(base, RPW)])
    pltpu.sync_copy(h1_v, out_hbm.at[pl.ds(NP + base, RPW)])


def kernel(x, edge_index, edge_weight):
    n, d = x.shape
    src = edge_index[0]
    dst = edge_index[1]
    e = src.shape[0]

    x_cat = jnp.zeros((2 * NP, DH), x.dtype)
    x_cat = x_cat.at[:n].set(x[:, :DH]).at[NP:NP + n].set(x[:, DH:])

    pad = EP - e
    fill = jnp.arange(pad, dtype=jnp.int32) % n
    src_p = jnp.concatenate([src, fill]).reshape(NS, C_CHUNKS, CHUNK)
    dst_p = jnp.concatenate([dst, fill]).reshape(NS, C_CHUNKS, CHUNK)
    w_p = jnp.concatenate(
        [edge_weight, jnp.zeros((pad,), edge_weight.dtype)]
    ).reshape(NS, C_CHUNKS, CHUNK)

    h = x_cat
    for _ in range(2):
        sup = _spmm(h, src_p, dst_p, w_p)
        h = _norm(sup)
    return jnp.concatenate([h[:n], h[NP:NP + n]], axis=1)
